# baseline (speedup 1.0000x reference)
import functools

import jax
import jax.numpy as jnp
from jax import lax
from jax.experimental import pallas as pl
from jax.experimental.pallas import tpu as pltpu, tpu_sc as plsc


@functools.lru_cache(maxsize=None)
def _make_gather(total_rows: int, dim: int):
    info = plsc.get_sparse_core_info()
    num_cores, num_subcores = info.num_cores, info.num_subcores
    num_workers = num_cores * num_subcores
    assert total_rows % (8 * num_workers) == 0
    rows_per_worker = total_rows // num_workers
    chunk = 416
    assert rows_per_worker % chunk == 0
    num_chunks = rows_per_worker // chunk
    mesh = plsc.VectorSubcoreMesh(core_axis_name="c", subcore_axis_name="s")

    @functools.partial(
        pl.kernel,
        mesh=mesh,
        out_type=jax.ShapeDtypeStruct((total_rows, dim), jnp.float32),
        compiler_params=pltpu.CompilerParams(use_tc_tiling_on_sc=False),
        scratch_types=[
            pltpu.VMEM((rows_per_worker,), jnp.int32),
            pltpu.VMEM((chunk, dim), jnp.float32),
            pltpu.VMEM((chunk, dim), jnp.float32),
            pltpu.SemaphoreType.DMA,
            pltpu.SemaphoreType.DMA,
        ],
    )
    def gather_kernel(idx_hbm, table_hbm, out_hbm, idx_v, buf0, buf1,
                      g_sem, o_sem):
        wid = lax.axis_index("s") * num_cores + lax.axis_index("c")
        base = wid * rows_per_worker
        pltpu.sync_copy(idx_hbm.at[pl.ds(base, rows_per_worker)], idx_v)
        bufs = (buf0, buf1)
        for c in range(num_chunks):
            buf = bufs[c % 2]
            pltpu.async_copy(
                table_hbm.at[idx_v.at[pl.ds(c * chunk, chunk)]], buf, g_sem
            ).wait()
            pltpu.sync_copy(buf, out_hbm.at[pl.ds(base + c * chunk, chunk)])

    return gather_kernel


def kernel(input, embedding_matrix):
    batch, fields = input.shape
    _, dim = embedding_matrix.shape
    idx = input.reshape(-1).astype(jnp.int32)
    out = _make_gather(batch * fields, dim)(idx, embedding_matrix)
    return out.reshape(batch, fields, dim)


# speedup vs baseline: 1.0054x; 1.0054x over previous
import functools

import jax
import jax.numpy as jnp
from jax import lax
from jax.experimental import pallas as pl
from jax.experimental.pallas import tpu as pltpu, tpu_sc as plsc


@functools.lru_cache(maxsize=None)
def _make_gather(total_rows: int, dim: int):
    info = plsc.get_sparse_core_info()
    num_cores, num_subcores = info.num_cores, info.num_subcores
    num_workers = num_cores * num_subcores
    assert total_rows % (8 * num_workers) == 0
    rows_per_worker = total_rows // num_workers
    chunk = 416
    nbuf = 4
    assert rows_per_worker % chunk == 0
    num_chunks = rows_per_worker // chunk
    mesh = plsc.VectorSubcoreMesh(core_axis_name="c", subcore_axis_name="s")

    @functools.partial(
        pl.kernel,
        mesh=mesh,
        out_type=jax.ShapeDtypeStruct((total_rows, dim), jnp.float32),
        compiler_params=pltpu.CompilerParams(use_tc_tiling_on_sc=False),
        scratch_types=[
            pltpu.VMEM((rows_per_worker,), jnp.int32),
            [pltpu.VMEM((chunk, dim), jnp.float32) for _ in range(nbuf)],
            [pltpu.SemaphoreType.DMA for _ in range(nbuf)],
            [pltpu.SemaphoreType.DMA for _ in range(nbuf)],
        ],
    )
    def gather_kernel(idx_hbm, table_hbm, out_hbm, idx_v, bufs, g_sems,
                      o_sems):
        wid = lax.axis_index("s") * num_cores + lax.axis_index("c")
        base = wid * rows_per_worker
        pltpu.sync_copy(idx_hbm.at[pl.ds(base, rows_per_worker)], idx_v)

        def start_gather(c):
            return pltpu.async_copy(
                table_hbm.at[idx_v.at[pl.ds(c * chunk, chunk)]],
                bufs[c % nbuf],
                g_sems[c % nbuf],
            )

        def start_out(c):
            return pltpu.async_copy(
                bufs[c % nbuf],
                out_hbm.at[pl.ds(base + c * chunk, chunk)],
                o_sems[c % nbuf],
            )

        gd = {}
        od = {}
        for c in range(min(nbuf, num_chunks)):
            gd[c] = start_gather(c)
        for c in range(num_chunks):
            gd[c].wait()
            od[c] = start_out(c)
            nxt = c + nbuf
            if nxt < num_chunks:
                od[c].wait()
                gd[nxt] = start_gather(nxt)
        for c in range(max(0, num_chunks - nbuf), num_chunks):
            od[c].wait()

    return gather_kernel


def kernel(input, embedding_matrix):
    batch, fields = input.shape
    _, dim = embedding_matrix.shape
    idx = input.reshape(-1).astype(jnp.int32)
    out = _make_gather(batch * fields, dim)(idx, embedding_matrix)
    return out.reshape(batch, fields, dim)


# speedup vs baseline: 1.4650x; 1.4572x over previous
import functools

import jax
import jax.numpy as jnp
from jax import lax
from jax.experimental import pallas as pl
from jax.experimental.pallas import tpu as pltpu, tpu_sc as plsc

_CH = 64
_NSLOT = 4


@functools.lru_cache(maxsize=None)
def _make_lookup(vocab: int, total_rows: int, dim: int):
    info = plsc.get_sparse_core_info()
    num_cores, num_subcores = info.num_cores, info.num_subcores
    num_workers = num_cores * num_subcores
    assert total_rows % (_CH * num_workers) == 0
    rows_per_worker = total_rows // num_workers
    num_chunks = rows_per_worker // _CH
    mesh = plsc.VectorSubcoreMesh(core_axis_name="c", subcore_axis_name="s")

    @functools.partial(
        pl.kernel,
        mesh=mesh,
        out_type=jax.ShapeDtypeStruct((total_rows, dim), jnp.float32),
        compiler_params=pltpu.CompilerParams(
            use_tc_tiling_on_sc=True, needs_layout_passes=False
        ),
        scratch_types=[
            pltpu.VMEM((rows_per_worker,), jnp.int32),
            pltpu.VMEM((_NSLOT * _CH, dim), jnp.float32),
            [pltpu.SemaphoreType.DMA for _ in range(_NSLOT)],
            [pltpu.SemaphoreType.DMA for _ in range(_NSLOT)],
        ],
    )
    def lookup_kernel(idx_hbm, tab_hbm, out_hbm, iv, tbuf, g_sems, o_sems):
        wid = lax.axis_index("s") * num_cores + lax.axis_index("c")
        base = wid * rows_per_worker
        pltpu.sync_copy(idx_hbm.at[pl.ds(base, rows_per_worker)], iv)

        def fire_rows(c, slot):
            def grp(g, carry):
                v16 = iv[pl.ds(c * _CH + g * 16, 16)]
                for j in range(16):
                    pltpu.async_copy(
                        tab_hbm.at[pl.ds(v16[j], 1)],
                        tbuf.at[pl.ds(slot * _CH + g * 16 + j, 1)],
                        g_sems[slot],
                    )
                return carry

            lax.fori_loop(0, _CH // 16, grp, 0)

        def drain_gathers(slot):
            pltpu.make_async_copy(
                tab_hbm.at[pl.ds(0, _CH)],
                tbuf.at[pl.ds(slot * _CH, _CH)],
                g_sems[slot],
            ).wait()

        def writeback(c, slot):
            return pltpu.async_copy(
                tbuf.at[pl.ds(slot * _CH, _CH)],
                out_hbm.at[pl.ds(base + c * _CH, _CH)],
                o_sems[slot],
            )

        def wait_writeback(slot):
            pltpu.make_async_copy(
                tbuf.at[pl.ds(slot * _CH, _CH)],
                out_hbm.at[pl.ds(base, _CH)],
                o_sems[slot],
            ).wait()

        fire_rows(0, 0)

        def body(c, carry):
            for k in range(_NSLOT):
                @pl.when(lax.rem(c, _NSLOT) == k)
                def _():
                    kn = (k + 1) % _NSLOT
                    @pl.when(c >= _NSLOT - 1)
                    def _():
                        wait_writeback(kn)

                    @pl.when(c < num_chunks - 1)
                    def _():
                        fire_rows(c + 1, kn)

                    drain_gathers(k)
                    writeback(c, k)
            return carry

        lax.fori_loop(0, num_chunks, body, 0)

        for last in range(num_chunks - _NSLOT + 1, num_chunks):
            wait_writeback(last % _NSLOT)

    return lookup_kernel


def kernel(input, embedding_matrix):
    batch, fields = input.shape
    vocab, dim = embedding_matrix.shape
    idx = input.reshape(-1).astype(jnp.int32)
    out = _make_lookup(vocab, batch * fields, dim)(idx, embedding_matrix)
    return out.reshape(batch, fields, dim)


# speedup vs baseline: 1.4653x; 1.0002x over previous
import functools

import jax
import jax.numpy as jnp
from jax import lax
from jax.experimental import pallas as pl
from jax.experimental.pallas import tpu as pltpu, tpu_sc as plsc

_CH = 64
_NSLOT = 4


@functools.lru_cache(maxsize=None)
def _make_lookup(vocab: int, total_rows: int, dim: int):
    info = plsc.get_sparse_core_info()
    num_cores, num_subcores = info.num_cores, info.num_subcores
    num_workers = num_cores * num_subcores
    assert total_rows % (_CH * num_workers) == 0
    rows_per_worker = total_rows // num_workers
    num_chunks = rows_per_worker // _CH
    mesh = plsc.VectorSubcoreMesh(core_axis_name="c", subcore_axis_name="s")

    @functools.partial(
        pl.kernel,
        mesh=mesh,
        out_type=jax.ShapeDtypeStruct((total_rows, dim), jnp.float32),
        compiler_params=pltpu.CompilerParams(use_tc_tiling_on_sc=True),
        scratch_types=[
            pltpu.VMEM((rows_per_worker,), jnp.int32),
            pltpu.VMEM((_NSLOT * _CH, dim), jnp.float32),
            [pltpu.SemaphoreType.DMA for _ in range(_NSLOT)],
            [pltpu.SemaphoreType.DMA for _ in range(_NSLOT)],
        ],
    )
    def lookup_kernel(idx_hbm, tab_hbm, out_hbm, iv, tbuf, g_sems, o_sems):
        wid = lax.axis_index("s") * num_cores + lax.axis_index("c")
        base = wid * rows_per_worker
        pltpu.sync_copy(idx_hbm.at[pl.ds(base, rows_per_worker)], iv)

        def fire_rows(c, slot):
            def grp(g, carry):
                v16 = iv[pl.ds(c * _CH + g * 16, 16)]
                for j in range(16):
                    pltpu.async_copy(
                        tab_hbm.at[pl.ds(v16[j], 1)],
                        tbuf.at[pl.ds(slot * _CH + g * 16 + j, 1)],
                        g_sems[slot],
                    )
                return carry

            lax.fori_loop(0, _CH // 16, grp, 0)

        def drain_gathers(slot):
            pltpu.make_async_copy(
                tab_hbm.at[pl.ds(0, _CH)],
                tbuf.at[pl.ds(slot * _CH, _CH)],
                g_sems[slot],
            ).wait()

        def writeback(c, slot):
            return pltpu.async_copy(
                tbuf.at[pl.ds(slot * _CH, _CH)],
                out_hbm.at[pl.ds(base + c * _CH, _CH)],
                o_sems[slot],
            )

        def wait_writeback(slot):
            pltpu.make_async_copy(
                tbuf.at[pl.ds(slot * _CH, _CH)],
                out_hbm.at[pl.ds(base, _CH)],
                o_sems[slot],
            ).wait()

        fire_rows(0, 0)

        def body(c, carry):
            for k in range(_NSLOT):
                @pl.when(lax.rem(c, _NSLOT) == k)
                def _():
                    kn = (k + 1) % _NSLOT
                    @pl.when(c >= _NSLOT - 1)
                    def _():
                        wait_writeback(kn)

                    @pl.when(c < num_chunks - 1)
                    def _():
                        fire_rows(c + 1, kn)

                    drain_gathers(k)
                    writeback(c, k)
            return carry

        lax.fori_loop(0, num_chunks, body, 0)

        for last in range(num_chunks - _NSLOT + 1, num_chunks):
            wait_writeback(last % _NSLOT)

    return lookup_kernel


def kernel(input, embedding_matrix):
    batch, fields = input.shape
    vocab, dim = embedding_matrix.shape
    idx = input.reshape(-1).astype(jnp.int32)
    out = _make_lookup(vocab, batch * fields, dim)(idx, embedding_matrix)
    return out.reshape(batch, fields, dim)


# speedup vs baseline: 1.5986x; 1.0909x over previous
import functools

import jax
import jax.numpy as jnp
from jax import lax
from jax.experimental import pallas as pl
from jax.experimental.pallas import tpu as pltpu, tpu_sc as plsc

_CB = 8
_NSLOT = 2


@functools.lru_cache(maxsize=None)
def _make_lookup(vocab: int, batch: int, fields: int, dim: int):
    info = plsc.get_sparse_core_info()
    num_cores, num_subcores = info.num_cores, info.num_subcores
    num_workers = num_cores * num_subcores
    assert batch % (_CB * num_workers) == 0
    b_per_worker = batch // num_workers
    rows_per_worker = b_per_worker * fields
    num_groups = rows_per_worker // 16
    num_chunks = b_per_worker // _CB
    ch_rows = _CB * fields
    mesh = plsc.VectorSubcoreMesh(core_axis_name="c", subcore_axis_name="s")

    @functools.partial(
        pl.kernel,
        mesh=mesh,
        out_type=jax.ShapeDtypeStruct((batch, fields, dim), jnp.float32),
        compiler_params=pltpu.CompilerParams(use_tc_tiling_on_sc=True),
        scratch_types=[
            pltpu.VMEM((rows_per_worker,), jnp.int32),
            pltpu.VMEM((rows_per_worker,), jnp.int32),
            pltpu.VMEM((rows_per_worker,), jnp.int32),
            pltpu.VMEM((_NSLOT * _CB, fields, dim), jnp.float32),
            [pltpu.SemaphoreType.DMA for _ in range(_NSLOT)],
            [pltpu.SemaphoreType.DMA for _ in range(_NSLOT)],
        ],
    )
    def lookup_kernel(idx_hbm, tab_hbm, out_hbm, iv, qb, fb, tbuf,
                      g_sems, o_sems):
        wid = lax.axis_index("s") * num_cores + lax.axis_index("c")
        base = wid * rows_per_worker
        bbase = wid * b_per_worker
        pltpu.sync_copy(idx_hbm.at[pl.ds(base, rows_per_worker)], iv)

        iota = lax.iota(jnp.int32, 16)

        def prep(g, carry):
            r = g * 16 + iota
            qb[pl.ds(g * 16, 16)] = lax.div(r, fields)
            fb[pl.ds(g * 16, 16)] = lax.rem(r, fields)
            return carry

        lax.fori_loop(0, num_groups, prep, 0)

        def fire_rows(c, slot):
            def grp(g, carry):
                off = c * ch_rows + g * 16
                v16 = iv[pl.ds(off, 16)]
                q16 = qb[pl.ds(off, 16)]
                f16 = fb[pl.ds(off, 16)]
                for j in range(16):
                    pltpu.async_copy(
                        tab_hbm.at[pl.ds(v16[j], 1)],
                        tbuf.at[slot * _CB + q16[j] - c * _CB].at[
                            pl.ds(f16[j], 1)
                        ],
                        g_sems[slot],
                    )
                return carry

            lax.fori_loop(0, ch_rows // 16, grp, 0)

        def drain_gathers(slot):
            pltpu.make_async_copy(
                out_hbm.at[pl.ds(0, _CB)],
                tbuf.at[pl.ds(slot * _CB, _CB)],
                g_sems[slot],
            ).wait()

        def writeback(c, slot):
            return pltpu.async_copy(
                tbuf.at[pl.ds(slot * _CB, _CB)],
                out_hbm.at[pl.ds(bbase + c * _CB, _CB)],
                o_sems[slot],
            )

        def wait_writeback(slot):
            pltpu.make_async_copy(
                tbuf.at[pl.ds(slot * _CB, _CB)],
                out_hbm.at[pl.ds(bbase, _CB)],
                o_sems[slot],
            ).wait()

        fire_rows(0, 0)

        def body(c, carry):
            for k in range(_NSLOT):
                @pl.when(lax.rem(c, _NSLOT) == k)
                def _():
                    kn = (k + 1) % _NSLOT
                    @pl.when(c >= _NSLOT - 1)
                    def _():
                        wait_writeback(kn)

                    @pl.when(c < num_chunks - 1)
                    def _():
                        fire_rows(c + 1, kn)

                    drain_gathers(k)
                    writeback(c, k)
            return carry

        lax.fori_loop(0, num_chunks, body, 0)

        for last in range(num_chunks - _NSLOT + 1, num_chunks):
            wait_writeback(last % _NSLOT)

    return lookup_kernel


def kernel(input, embedding_matrix):
    batch, fields = input.shape
    vocab, dim = embedding_matrix.shape
    idx = input.reshape(-1).astype(jnp.int32)
    return _make_lookup(vocab, batch, fields, dim)(idx, embedding_matrix)
